# baseline (speedup 1.0000x reference)
import functools

import jax
import jax.numpy as jnp
from jax import lax
from jax.experimental import pallas as pl
from jax.experimental.pallas import tpu as pltpu
from jax.experimental.pallas import tpu_sc as plsc

B = 4096
L = 200
EMB = 64
TOTAL = B * L
NUM_CORES = 2
NUM_SUBCORES = 16
NW = NUM_CORES * NUM_SUBCORES
ROWS_W = B // NW
SPLIT = 128
R = ROWS_W
NB = 4

_mesh = plsc.VectorSubcoreMesh(core_axis_name="c", subcore_axis_name="s")


@functools.partial(
    pl.kernel,
    out_type=jax.ShapeDtypeStruct((TOTAL, EMB), jnp.float32),
    mesh=_mesh,
    scratch_types=(
        [pltpu.VMEM((ROWS_W, L), jnp.int32)]
        + [pltpu.VMEM((L, EMB), jnp.float32)
           for _ in range(NB)]
        + [pltpu.SemaphoreType.DMA for _ in range(2 * NB)]
    ),
    compiler_params=pltpu.CompilerParams(use_tc_tiling_on_sc=False),
)
def _emb_lookup(idx_hbm, table_hbm, out_hbm, idx_v,
                bank0, bank1, bank2, bank3,
                g0, g1, g2, g3, o0, o1, o2, o3):
    banks = (bank0, bank1, bank2, bank3)
    gsems = (g0, g1, g2, g3)
    osems = (o0, o1, o2, o3)

    wid = lax.axis_index("s") * NUM_CORES + lax.axis_index("c")
    pltpu.sync_copy(idx_hbm.at[pl.ds(wid * ROWS_W, ROWS_W)], idx_v)
    base = wid * ROWS_W * L

    def fire_gathers(r, bi):
        pltpu.async_copy(table_hbm.at[idx_v.at[r, pl.ds(0, SPLIT)]],
                         banks[bi].at[pl.ds(0, SPLIT)], gsems[bi])
        pltpu.async_copy(table_hbm.at[idx_v.at[r, pl.ds(SPLIT, L - SPLIT)]],
                         banks[bi].at[pl.ds(SPLIT, L - SPLIT)], gsems[bi])

    def drain_gathers(bi):
        pltpu.make_async_copy(table_hbm.at[pl.ds(0, L)],
                              banks[bi], gsems[bi]).wait()

    def fire_write(r, bi):
        pltpu.async_copy(banks[bi],
                         out_hbm.at[pl.ds(base + r * L, L)], osems[bi])

    def drain_write(bi):
        pltpu.make_async_copy(banks[bi],
                              out_hbm.at[pl.ds(0, L)], osems[bi]).wait()

    def do_round(r, bi, drain_w=True, fire_g=True):
        ob = (bi + 2) % NB
        if drain_w:
            drain_write(ob)
        if fire_g:
            fire_gathers(r + 2, ob)
        drain_gathers(bi)
        fire_write(r, bi)

    fire_gathers(0, 0)
    fire_gathers(1, 1)
    do_round(0, 0, drain_w=False)
    do_round(1, 1, drain_w=False)

    @pl.loop(2, R - 2, step=NB)
    def _rounds(r0):
        do_round(r0, 2)
        do_round(r0 + 1, 3)
        do_round(r0 + 2, 0)
        do_round(r0 + 3, 1)

    do_round(R - 2, 2, fire_g=False)
    do_round(R - 1, 3, fire_g=False)
    drain_write(2)
    drain_write(3)


def kernel(x, table):
    out = _emb_lookup(x.astype(jnp.int32), table)
    return out.reshape(B, L, EMB)


# speedup vs baseline: 1.1701x; 1.1701x over previous
import functools

import jax
import jax.numpy as jnp
from jax import lax
from jax.experimental import pallas as pl
from jax.experimental.pallas import tpu as pltpu
from jax.experimental.pallas import tpu_sc as plsc

B = 4096
L = 200
EMB = 64
NUM_CORES = 2
NUM_SUBCORES = 16
NW = NUM_CORES * NUM_SUBCORES
CHUNK = 128
NB = 3
TPITCH = 129

_mesh = plsc.VectorSubcoreMesh(core_axis_name="c", subcore_axis_name="s")


@functools.partial(
    pl.kernel,
    out_type=jax.ShapeDtypeStruct((L, EMB // 8, NW, 8, CHUNK), jnp.float32),
    mesh=_mesh,
    scratch_types=(
        [pltpu.VMEM((L, CHUNK), jnp.int32)]
        + [pltpu.VMEM((CHUNK, EMB), jnp.float32)
           for _ in range(NB)]
        + [pltpu.VMEM((EMB // 8, 8, TPITCH), jnp.float32)
           for _ in range(2)]
        + [pltpu.SemaphoreType.DMA for _ in range(NB + 2)]
    ),
    compiler_params=pltpu.CompilerParams(use_tc_tiling_on_sc=False, needs_layout_passes=False),
)
def _emb_lookup(xt_hbm, table_hbm, out_hbm, idx_v,
                bank0, bank1, bank2, tb0, tb1,
                g0, g1, g2, o0, o1):
    banks = (bank0, bank1, bank2)
    tbufs = (tb0, tb1)
    gsems = (g0, g1, g2)
    osems = (o0, o1)

    wid = lax.axis_index("s") * NUM_CORES + lax.axis_index("c")
    pltpu.sync_copy(xt_hbm.at[:, pl.ds(wid * CHUNK, CHUNK)], idx_v)

    def fire_gather(r, bi):
        pltpu.async_copy(table_hbm.at[idx_v.at[r]], banks[bi], gsems[bi])

    def drain_gather(bi):
        pltpu.make_async_copy(table_hbm.at[pl.ds(0, CHUNK)],
                              banks[bi], gsems[bi]).wait()

    def transpose(bi, ti):
        bank, tbuf = banks[bi], tbufs[ti]

        @pl.loop(0, CHUNK)
        def _rows(b):
            lane = jnp.full((16,), b, jnp.int32)
            for j in range(EMB // 16):
                e = lax.iota(jnp.int32, 16) + (j * 16)
                vals = plsc.load_gather(bank, [lane, e])
                plsc.store_scatter(tbuf, [e >> 3, e & 7, lane], vals)

    def fire_write(r, ti):
        pltpu.async_copy(tbufs[ti].at[:, :, pl.ds(0, CHUNK)],
                         out_hbm.at[r, :, wid], osems[ti])

    def drain_write(ti):
        pltpu.make_async_copy(tbufs[ti].at[:, :, pl.ds(0, CHUNK)],
                              out_hbm.at[0, :, 0], osems[ti]).wait()

    def do_round(r, bi, ti, drain_w=True, fire_g=True):
        if fire_g:
            fire_gather(r + 2, (bi + 2) % NB)
        drain_gather(bi)
        if drain_w:
            drain_write(ti)
        transpose(bi, ti)
        fire_write(r, ti)

    fire_gather(0, 0)
    fire_gather(1, 1)
    do_round(0, 0, 0, drain_w=False)
    do_round(1, 1, 1, drain_w=False)

    @pl.loop(2, L - 6, step=6)
    def _rounds(r0):
        for k in range(6):
            do_round(r0 + k, (2 + k) % NB, k % 2)

    for k in range(6):
        r = L - 6 + k
        do_round(r, r % NB, r % 2, fire_g=(r + 2 < L))
    drain_write(0)
    drain_write(1)


def kernel(x, table):
    out = _emb_lookup(x.T.astype(jnp.int32), table)
    return out.transpose(2, 4, 0, 1, 3).reshape(B, L, EMB)


# speedup vs baseline: 1.2393x; 1.0591x over previous
import functools

import jax
import jax.numpy as jnp
from jax import lax
from jax.experimental import pallas as pl
from jax.experimental.pallas import tpu as pltpu
from jax.experimental.pallas import tpu_sc as plsc

B = 4096
L = 200
EMB = 64
NUM_CORES = 2
NUM_SUBCORES = 16
NW = NUM_CORES * NUM_SUBCORES
CHUNK = 128
NB = 3
TPITCH = 129

_mesh = plsc.VectorSubcoreMesh(core_axis_name="c", subcore_axis_name="s")


@functools.partial(
    pl.kernel,
    out_type=jax.ShapeDtypeStruct((L, EMB // 8, NW, 8, CHUNK), jnp.float32),
    mesh=_mesh,
    scratch_types=(
        [pltpu.VMEM((L, CHUNK), jnp.int32)]
        + [pltpu.VMEM((CHUNK, EMB), jnp.float32)
           for _ in range(NB)]
        + [pltpu.VMEM((EMB // 8, 8, TPITCH), jnp.float32)
           for _ in range(2)]
        + [pltpu.SemaphoreType.DMA for _ in range(NB + 2)]
    ),
    compiler_params=pltpu.CompilerParams(use_tc_tiling_on_sc=False, needs_layout_passes=False),
)
def _emb_lookup(xt_hbm, table_hbm, out_hbm, idx_v,
                bank0, bank1, bank2, tb0, tb1,
                g0, g1, g2, o0, o1):
    banks = (bank0, bank1, bank2)
    tbufs = (tb0, tb1)
    gsems = (g0, g1, g2)
    osems = (o0, o1)

    wid = lax.axis_index("s") * NUM_CORES + lax.axis_index("c")
    pltpu.sync_copy(xt_hbm.at[:, pl.ds(wid * CHUNK, CHUNK)], idx_v)

    def fire_gather(r, bi):
        pltpu.async_copy(table_hbm.at[idx_v.at[r]], banks[bi], gsems[bi])

    def drain_gather(bi):
        pltpu.make_async_copy(table_hbm.at[pl.ds(0, CHUNK)],
                              banks[bi], gsems[bi]).wait()

    _es = [lax.iota(jnp.int32, 16) + 16 * j for j in range(EMB // 16)]
    _ers = [e >> 3 for e in _es]
    _ris = [e & 7 for e in _es]

    def transpose(bi, ti):
        bank, tbuf = banks[bi], tbufs[ti]

        @pl.loop(0, CHUNK, unroll=8)
        def _rows(b):
            lane = jnp.full((16,), b, jnp.int32)
            for j in range(EMB // 16):
                vals = bank[b, pl.ds(j * 16, 16)]
                plsc.store_scatter(tbuf, [_ers[j], _ris[j], lane], vals)

    def fire_write(r, ti):
        pltpu.async_copy(tbufs[ti].at[:, :, pl.ds(0, CHUNK)],
                         out_hbm.at[r, :, wid], osems[ti])

    def drain_write(ti):
        pltpu.make_async_copy(tbufs[ti].at[:, :, pl.ds(0, CHUNK)],
                              out_hbm.at[0, :, 0], osems[ti]).wait()

    def do_round(r, bi, ti, drain_w=True, fire_g=True):
        if fire_g:
            fire_gather(r + 2, (bi + 2) % NB)
        drain_gather(bi)
        if drain_w:
            drain_write(ti)
        transpose(bi, ti)
        fire_write(r, ti)

    fire_gather(0, 0)
    fire_gather(1, 1)
    do_round(0, 0, 0, drain_w=False)
    do_round(1, 1, 1, drain_w=False)

    @pl.loop(2, L - 6, step=6)
    def _rounds(r0):
        for k in range(6):
            do_round(r0 + k, (2 + k) % NB, k % 2)

    for k in range(6):
        r = L - 6 + k
        do_round(r, r % NB, r % 2, fire_g=(r + 2 < L))
    drain_write(0)
    drain_write(1)


def kernel(x, table):
    out = _emb_lookup(x.T.astype(jnp.int32), table)
    return out.transpose(2, 4, 0, 1, 3).reshape(B, L, EMB)
